# baseline (speedup 1.0000x reference)
import functools
import jax
import jax.numpy as jnp
from jax.experimental import pallas as pl

N = 10000
E = 160000
D = 256
R = 50
NUM_LAYER = 4
NODE_RATIO = 0.1
NEG = 33

BN = 200


def _dense_tail_body(mean_ref, mx_ref, mn_ref, std_ref, amp_ref, att_ref,
                     msk_ref, hid_ref, wout_ref, bout_ref, w1_ref, b1_ref,
                     w2_ref, b2_ref, hid_out_ref, score_out_ref):
    has_edge = msk_ref[...] > 0.0
    aggs = jnp.concatenate(
        [mean_ref[...], mx_ref[...], mn_ref[...], std_ref[...]], axis=1)
    feats = jnp.concatenate(
        [aggs, aggs * amp_ref[...], aggs * att_ref[...]], axis=1)

    hidden_out = jnp.dot(feats, wout_ref[...],
                         preferred_element_type=jnp.float32) + bout_ref[...]
    hidden_new = hid_ref[...] + jnp.where(has_edge, hidden_out, 0.0)
    hid_out_ref[...] = hidden_new

    t = jnp.maximum(
        jnp.dot(hidden_new, w1_ref[...], preferred_element_type=jnp.float32)
        + b1_ref[...], 0.0)
    score = jnp.dot(t, w2_ref[...], preferred_element_type=jnp.float32) \
        + b2_ref[...]
    score_out_ref[...] = score


@jax.jit
def _dense_tail(mean, mx, mn, std, amp, att, msk, hidden,
                w_out_i, b_out_i, w1, b1, w2, b2):
    grid = (N // BN,)
    row = pl.BlockSpec((BN, D), lambda i: (i, 0))
    col1 = pl.BlockSpec((BN, 1), lambda i: (i, 0))
    full = lambda shape: pl.BlockSpec(shape, lambda i: (0, 0))
    return pl.pallas_call(
        _dense_tail_body,
        grid=grid,
        in_specs=[row, row, row, row, col1, col1, col1, row,
                  full((12 * D, D)), full((1, D)),
                  full((D, 2 * D)), full((1, 2 * D)),
                  full((2 * D, 1)), full((1, 1))],
        out_specs=[row, col1],
        out_shape=[jax.ShapeDtypeStruct((N, D), jnp.float32),
                   jax.ShapeDtypeStruct((N, 1), jnp.float32)],
    )(mean, mx, mn, std, amp, att, msk, hidden,
      w_out_i, b_out_i, w1, b1, w2, b2)


def kernel(h_index, r_index, t_index, hidden_states, rel_hidden_states,
           edge_index, edge_attr, score_text_embs, all_index,
           rel_embedding, msg_rel, W_out, b_out, W1, b1, W2, b2):
    src = jnp.concatenate([edge_index[0], edge_index[1]], axis=0)
    dst = jnp.concatenate([edge_index[1], edge_index[0]], axis=0)
    ea = jnp.concatenate([edge_attr, edge_attr + R], axis=0)
    E2 = 2 * E

    r0 = r_index[:, 0]
    rel_embeds = rel_embedding[r0] + rel_hidden_states[r0]

    boundary = jnp.zeros((N, D), dtype=jnp.float32)
    boundary = boundary.at[h_index[:, 0]].add(rel_embeds + hidden_states)
    boundary = boundary.at[all_index].add(score_text_embs)
    init_score = jnp.zeros((N,), dtype=jnp.float32).at[h_index[:, 0]].set(5.0)

    degree_out = jax.ops.segment_sum(jnp.ones((E2,), jnp.float32), src,
                                     num_segments=N)
    pna_mean = jnp.log(degree_out + 1.0).mean()

    hidden = boundary
    score = init_score
    k_sel = int(NODE_RATIO * E2)

    for i in range(NUM_LAYER):
        edge_scores = score[src]
        _, top_idx = jax.lax.top_k(edge_scores, k_sel)
        s_src = src[top_idx]
        s_dst = dst[top_idx]
        s_ea = ea[top_idx]

        layer_input = jax.nn.sigmoid(score)[:, None] * hidden
        msg = layer_input[s_src] * msg_rel[i][s_ea]

        deg = jax.ops.segment_sum(jnp.ones((k_sel,), jnp.float32), s_dst,
                                  num_segments=N)
        deg_safe = jnp.clip(deg, 1.0, None)
        has_edge = deg > 0.0
        agg_sum = jax.ops.segment_sum(msg, s_dst, num_segments=N)
        mean = agg_sum / deg_safe[:, None]
        agg_max = jax.ops.segment_max(msg, s_dst, num_segments=N)
        mx = jnp.where(has_edge[:, None], agg_max, 0.0)
        agg_min = -jax.ops.segment_max(-msg, s_dst, num_segments=N)
        mn = jnp.where(has_edge[:, None], agg_min, 0.0)
        agg_sq = jax.ops.segment_sum(msg * msg, s_dst,
                                     num_segments=N) / deg_safe[:, None]
        std = jnp.sqrt(jnp.clip(agg_sq - mean * mean, 0.0, None) + 1e-6)

        deg_l = jnp.log(deg + 1.0)
        amp = (deg_l / pna_mean)[:, None]
        att = jnp.where(deg > 0,
                        pna_mean / jnp.clip(deg_l, 1e-6, None), 0.0)[:, None]

        hidden, score2d = _dense_tail(
            mean, mx, mn, std, amp, att,
            has_edge.astype(jnp.float32).reshape(N, 1), hidden,
            W_out[i], b_out[i].reshape(1, D), W1, b1.reshape(1, 2 * D),
            W2, b2.reshape(1, 1))
        score = score2d[:, 0]

    return score[t_index]
